# baseline (speedup 1.0000x reference)
import functools

import jax
import jax.numpy as jnp
from jax import lax
from jax.experimental import pallas as pl
from jax.experimental.pallas import tpu as pltpu
from jax.experimental.pallas import tpu_sc as plsc

_TEMP = 0.1
_K = 8
_NC = 2
_NS = 16
_LANES = 16


def _logits_block(x_ref, w1_ref, b1_ref, w2_ref, b2_ref, o_ref, ot_ref):
    h = jnp.tanh(
        jnp.dot(x_ref[...], w1_ref[...], preferred_element_type=jnp.float32)
        + b1_ref[...]
    )
    logits = jnp.dot(h, w2_ref[...], preferred_element_type=jnp.float32) + b2_ref[...]
    o_ref[...] = logits
    ot_ref[...] = logits.T


def _tc_logits(x, W1, b1, W2, b2, bm=1024):
    n, d = x.shape
    hdim = W1.shape[1]
    n_e = W2.shape[1]
    return pl.pallas_call(
        _logits_block,
        grid=(n // bm,),
        in_specs=[
            pl.BlockSpec((bm, d), lambda i: (i, 0)),
            pl.BlockSpec((d, hdim), lambda i: (0, 0)),
            pl.BlockSpec((1, hdim), lambda i: (0, 0)),
            pl.BlockSpec((hdim, n_e), lambda i: (0, 0)),
            pl.BlockSpec((1, n_e), lambda i: (0, 0)),
        ],
        out_specs=[
            pl.BlockSpec((bm, n_e), lambda i: (i, 0)),
            pl.BlockSpec((n_e, bm), lambda i: (0, i)),
        ],
        out_shape=[
            jax.ShapeDtypeStruct((n, n_e), jnp.float32),
            jax.ShapeDtypeStruct((n_e, n), jnp.float32),
        ],
    )(x, W1, b1.reshape(1, hdim), W2, b2.reshape(1, n_e))


def _make_sc_select(n, n_e):
    rw = n // (_NC * _NS)
    groups = rw // _LANES

    _SORT8 = [
        (0, 1), (2, 3), (4, 5), (6, 7),
        (0, 2), (1, 3), (4, 6), (5, 7),
        (1, 2), (5, 6),
        (0, 4), (1, 5), (2, 6), (3, 7),
        (2, 4), (3, 5),
        (1, 2), (3, 4), (5, 6),
    ]
    _BITONIC8 = [
        (0, 4), (1, 5), (2, 6), (3, 7),
        (0, 2), (1, 3), (4, 6), (5, 7),
        (0, 1), (2, 3), (4, 5), (6, 7),
    ]

    def _ce(v, i, j):
        hi = jnp.maximum(v[i], v[j])
        v[j] = jnp.minimum(v[i], v[j])
        v[i] = hi

    def body(lt_hbm, thr_hbm, bud_hbm, max_hbm, buf, thr_v, bud_v, max_v):
        c = lax.axis_index("c")
        s = lax.axis_index("s")
        base = (s * _NC + c) * rw
        pltpu.sync_copy(lt_hbm.at[:, pl.ds(base, rw)], buf)

        def group_body(g, carry):
            off = g * _LANES
            top = None
            for ci in range(n_e // _K):
                v = [buf[ci * _K + u, pl.ds(off, _LANES)]
                     for u in range(_K)]
                for i, j in _SORT8:
                    _ce(v, i, j)
                if top is None:
                    top = v
                else:
                    v = [jnp.maximum(top[i], v[_K - 1 - i]) for i in range(_K)]
                    for i, j in _BITONIC8:
                        _ce(v, i, j)
                    top = v
            thr = top[_K - 1]
            cgt = (top[0] > thr).astype(jnp.int32)
            for i in range(1, _K - 1):
                cgt = cgt + (top[i] > thr).astype(jnp.int32)
            thr_v[pl.ds(off, _LANES)] = thr
            bud_v[pl.ds(off, _LANES)] = (_K - cgt).astype(jnp.float32)
            max_v[pl.ds(off, _LANES)] = top[0]
            return carry

        lax.fori_loop(0, groups, group_body, 0)
        pltpu.sync_copy(thr_v, thr_hbm.at[pl.ds(base, rw)])
        pltpu.sync_copy(bud_v, bud_hbm.at[pl.ds(base, rw)])
        pltpu.sync_copy(max_v, max_hbm.at[pl.ds(base, rw)])

    return pl.kernel(
        body,
        out_type=(
            jax.ShapeDtypeStruct((n,), jnp.float32),
            jax.ShapeDtypeStruct((n,), jnp.float32),
            jax.ShapeDtypeStruct((n,), jnp.float32),
        ),
        mesh=plsc.VectorSubcoreMesh(core_axis_name="c", subcore_axis_name="s"),
        scratch_types=[
            pltpu.VMEM((n_e, rw), jnp.float32),
            pltpu.VMEM((rw,), jnp.float32),
            pltpu.VMEM((rw,), jnp.float32),
            pltpu.VMEM((rw,), jnp.float32),
        ],
        compiler_params=pltpu.CompilerParams(needs_layout_passes=False),
    )


def _finalize_block(l_ref, thr_ref, bud_ref, max_ref, tri_ref, o_ref):
    l = l_ref[...]
    thr = thr_ref[...]
    e = jnp.exp((l - max_ref[...]) * (1.0 / _TEMP))
    gt = l > thr
    eq = l == thr
    eqf = eq.astype(jnp.float32)
    eq_before = jnp.dot(eqf, tri_ref[...], preferred_element_type=jnp.float32)
    keep = gt | (eq & (eq_before < bud_ref[...]))
    val = jnp.where(keep, e, 0.0)
    s_all = jnp.sum(e, axis=-1, keepdims=True)
    s_hard = jnp.sum(val, axis=-1, keepdims=True)
    o_ref[...] = val / (s_hard + 1e-9 * s_all)


def _tc_finalize(logits, thr, bud, mx, bm=2048):
    n, n_e = logits.shape
    tri = jnp.triu(jnp.ones((n_e, n_e), jnp.float32), k=1)
    return pl.pallas_call(
        _finalize_block,
        grid=(n // bm,),
        in_specs=[
            pl.BlockSpec((bm, n_e), lambda i: (i, 0)),
            pl.BlockSpec((bm, 1), lambda i: (i, 0)),
            pl.BlockSpec((bm, 1), lambda i: (i, 0)),
            pl.BlockSpec((bm, 1), lambda i: (i, 0)),
            pl.BlockSpec((n_e, n_e), lambda i: (0, 0)),
        ],
        out_specs=pl.BlockSpec((bm, n_e), lambda i: (i, 0)),
        out_shape=jax.ShapeDtypeStruct((n, n_e), jnp.float32),
    )(logits, thr.reshape(n, 1), bud.reshape(n, 1), mx.reshape(n, 1), tri)


def kernel(x, W1, b1, W2, b2, topk):
    del topk
    n = x.shape[0]
    n_e = W2.shape[1]
    h = n // 2
    sc = _make_sc_select(h, n_e)
    parts = []
    for i in range(2):
        xs = lax.slice(x, (i * h, 0), ((i + 1) * h, x.shape[1]))
        lg, lgt = _tc_logits(xs, W1, b1, W2, b2)
        thr, bud, mx = sc(lgt)
        parts.append((lg, thr, bud, mx))
    outs = [_tc_finalize(lg, thr, bud, mx) for lg, thr, bud, mx in parts]
    return jnp.concatenate(outs, axis=0)


# speedup vs baseline: 2.2958x; 2.2958x over previous
import functools

import jax
import jax.numpy as jnp
from jax import lax
from jax.experimental import pallas as pl
from jax.experimental.pallas import tpu as pltpu
from jax.experimental.pallas import tpu_sc as plsc

_TEMP = 0.1
_K = 8
_NC = 2
_NS = 16
_LANES = 16


def _logits_block(x_ref, w1_ref, b1_ref, w2_ref, b2_ref, o_ref, ot_ref):
    h = jnp.tanh(
        jnp.dot(x_ref[...], w1_ref[...], preferred_element_type=jnp.float32)
        + b1_ref[...]
    )
    logits = jnp.dot(h, w2_ref[...], preferred_element_type=jnp.float32) + b2_ref[...]
    o_ref[...] = logits
    ot_ref[...] = logits.T


def _tc_logits(x, W1, b1, W2, b2, bm=1024):
    n, d = x.shape
    hdim = W1.shape[1]
    n_e = W2.shape[1]
    return pl.pallas_call(
        _logits_block,
        grid=(n // bm,),
        in_specs=[
            pl.BlockSpec((bm, d), lambda i: (i, 0)),
            pl.BlockSpec((d, hdim), lambda i: (0, 0)),
            pl.BlockSpec((1, hdim), lambda i: (0, 0)),
            pl.BlockSpec((hdim, n_e), lambda i: (0, 0)),
            pl.BlockSpec((1, n_e), lambda i: (0, 0)),
        ],
        out_specs=[
            pl.BlockSpec((bm, n_e), lambda i: (i, 0)),
            pl.BlockSpec((n_e, bm), lambda i: (0, i)),
        ],
        out_shape=[
            jax.ShapeDtypeStruct((n, n_e), jnp.float32),
            jax.ShapeDtypeStruct((n_e, n), jnp.float32),
        ],
    )(x, W1, b1.reshape(1, hdim), W2, b2.reshape(1, n_e))


def _make_sc_select(n, n_e):
    rw = n // (_NC * _NS)
    groups = rw // _LANES

    _SORT8 = [
        (0, 1), (2, 3), (4, 5), (6, 7),
        (0, 2), (1, 3), (4, 6), (5, 7),
        (1, 2), (5, 6),
        (0, 4), (1, 5), (2, 6), (3, 7),
        (2, 4), (3, 5),
        (1, 2), (3, 4), (5, 6),
    ]
    _BITONIC8 = [
        (0, 4), (1, 5), (2, 6), (3, 7),
        (0, 2), (1, 3), (4, 6), (5, 7),
        (0, 1), (2, 3), (4, 5), (6, 7),
    ]

    def _ce(v, i, j):
        hi = jnp.maximum(v[i], v[j])
        v[j] = jnp.minimum(v[i], v[j])
        v[i] = hi

    def body(lt_hbm, thr_hbm, bud_hbm, max_hbm, buf, thr_v, bud_v, max_v):
        c = lax.axis_index("c")
        s = lax.axis_index("s")
        base = (s * _NC + c) * rw
        pltpu.sync_copy(lt_hbm.at[:, pl.ds(base, rw)], buf)

        def group_body(g, carry):
            off = g * _LANES
            top = None
            for ci in range(n_e // _K):
                v = [buf[ci * _K + u, pl.ds(off, _LANES)]
                     for u in range(_K)]
                for i, j in _SORT8:
                    _ce(v, i, j)
                if top is None:
                    top = v
                else:
                    v = [jnp.maximum(top[i], v[_K - 1 - i]) for i in range(_K)]
                    for i, j in _BITONIC8:
                        _ce(v, i, j)
                    top = v
            thr = top[_K - 1]
            cgt = (top[0] > thr).astype(jnp.int32)
            for i in range(1, _K - 1):
                cgt = cgt + (top[i] > thr).astype(jnp.int32)
            thr_v[pl.ds(off, _LANES)] = thr
            bud_v[pl.ds(off, _LANES)] = (_K - cgt).astype(jnp.float32)
            max_v[pl.ds(off, _LANES)] = top[0]
            return carry

        lax.fori_loop(0, groups, group_body, 0)
        pltpu.sync_copy(thr_v, thr_hbm.at[pl.ds(base, rw)])
        pltpu.sync_copy(bud_v, bud_hbm.at[pl.ds(base, rw)])
        pltpu.sync_copy(max_v, max_hbm.at[pl.ds(base, rw)])

    return pl.kernel(
        body,
        out_type=(
            jax.ShapeDtypeStruct((n,), jnp.float32),
            jax.ShapeDtypeStruct((n,), jnp.float32),
            jax.ShapeDtypeStruct((n,), jnp.float32),
        ),
        mesh=plsc.VectorSubcoreMesh(core_axis_name="c", subcore_axis_name="s"),
        scratch_types=[
            pltpu.VMEM((n_e, rw), jnp.float32),
            pltpu.VMEM((rw,), jnp.float32),
            pltpu.VMEM((rw,), jnp.float32),
            pltpu.VMEM((rw,), jnp.float32),
        ],
        compiler_params=pltpu.CompilerParams(needs_layout_passes=False),
    )


def _finalize_block(l_ref, thr_ref, bud_ref, max_ref, tri_ref, o_ref):
    l = l_ref[...]
    thr = thr_ref[...]
    e = jnp.exp((l - max_ref[...]) * (1.0 / _TEMP))
    gt = l > thr
    eq = l == thr
    eqf = eq.astype(jnp.float32)
    eq_before = jnp.dot(eqf, tri_ref[...], preferred_element_type=jnp.float32)
    keep = gt | (eq & (eq_before < bud_ref[...]))
    val = jnp.where(keep, e, 0.0)
    s_all = jnp.sum(e, axis=-1, keepdims=True)
    s_hard = jnp.sum(val, axis=-1, keepdims=True)
    o_ref[...] = val / (s_hard + 1e-9 * s_all)


def _tc_finalize(logits, thr, bud, mx, bm=4096):
    n, n_e = logits.shape
    tri = jnp.triu(jnp.ones((n_e, n_e), jnp.float32), k=1)
    return pl.pallas_call(
        _finalize_block,
        grid=(n // bm,),
        in_specs=[
            pl.BlockSpec((bm, n_e), lambda i: (i, 0)),
            pl.BlockSpec((bm, 1), lambda i: (i, 0)),
            pl.BlockSpec((bm, 1), lambda i: (i, 0)),
            pl.BlockSpec((bm, 1), lambda i: (i, 0)),
            pl.BlockSpec((n_e, n_e), lambda i: (0, 0)),
        ],
        out_specs=pl.BlockSpec((bm, n_e), lambda i: (i, 0)),
        out_shape=jax.ShapeDtypeStruct((n, n_e), jnp.float32),
    )(logits, thr.reshape(n, 1), bud.reshape(n, 1), mx.reshape(n, 1), tri)


def kernel(x, W1, b1, W2, b2, topk):
    del topk
    n = x.shape[0]
    n_e = W2.shape[1]
    logits, logits_t = _tc_logits(x, W1, b1, W2, b2)
    thr, bud, mx = _make_sc_select(n, n_e)(logits_t)
    return _tc_finalize(logits, thr, bud, mx)
